# Initial kernel scaffold; baseline (speedup 1.0000x reference)
import jax
import jax.numpy as jnp
from jax.experimental import pallas as pl


def kernel(feature_torch, edge_torch, i, j, W0, b0, W1, b1, fc1_W, fc1_b, fc2_W, fc2_b):
    raise NotImplementedError("write your pallas kernel here")


# speedup vs baseline: 72.7943x; 72.7943x over previous
import functools

import jax
import jax.numpy as jnp
from jax import lax
from jax.experimental import pallas as pl
from jax.experimental.pallas import tpu as pltpu
from jax.experimental.pallas import tpu_sc as plsc

_N = 100000
_E = 3200000
_NPAD = 102400
_EPAD = 3276800
_ROWS = _EPAD // 128
_RPT = _ROWS // 32
_NPT = _NPAD // 16
_B = 12800
_DUMMY = _N

_mesh = plsc.VectorSubcoreMesh(core_axis_name="c", subcore_axis_name="s",
                               num_cores=2)


def _wid(c, s):
    return c * 16 + s


@functools.partial(
    pl.kernel,
    mesh=_mesh,
    out_type=jax.ShapeDtypeStruct((2, _NPAD), jnp.float32),
    scratch_types=[
        pltpu.VMEM((16, 128), jnp.int32),
        pltpu.VMEM((128,), jnp.float32),
        pltpu.VMEM_SHARED((_NPAD,), jnp.float32),
        pltpu.SemaphoreType.DMA,
    ],
)
def _deg_sc(dst_hbm, z1_hbm, deg_out, dst_v, ones_v, deg_sh, sem):
    c = lax.axis_index("c")
    s = lax.axis_index("s")
    w = _wid(c, s)
    for k in range(8):
        ones_v[pl.ds(k * 16, 16)] = jnp.full((16,), 1.0, jnp.float32)
    sl = pl.ds(s * _NPT, _NPT)
    pltpu.sync_copy(z1_hbm.at[sl], deg_sh.at[sl])
    plsc.subcore_barrier()
    row0 = w * _RPT

    def outer(t, carry):
        pltpu.sync_copy(dst_hbm.at[pl.ds(row0 + t * 16, 16)], dst_v)

        def inner(g, carry2):
            pltpu.sync_copy(ones_v, deg_sh.at[dst_v.at[g]], add=True)
            return carry2

        lax.fori_loop(0, 16, inner, 0)
        return carry

    lax.fori_loop(0, _RPT // 16, outer, 0)
    plsc.subcore_barrier()
    pltpu.sync_copy(deg_sh.at[sl], deg_out.at[c, sl])


@functools.partial(
    pl.kernel,
    mesh=_mesh,
    out_type=[
        jax.ShapeDtypeStruct((2, _NPAD), jnp.float32),
        jax.ShapeDtypeStruct((2, _NPAD), jnp.float32),
        jax.ShapeDtypeStruct((2, _NPAD), jnp.float32),
        jax.ShapeDtypeStruct((2, _NPAD), jnp.float32),
    ],
    scratch_types=[
        pltpu.VMEM((16, 128), jnp.int32),
        pltpu.VMEM((16, 128), jnp.int32),
        pltpu.VMEM((128,), jnp.float32),
        pltpu.VMEM((128,), jnp.float32),
        pltpu.VMEM((128,), jnp.float32),
        pltpu.VMEM((128,), jnp.float32),
        pltpu.VMEM((16,), jnp.int32),
        pltpu.VMEM((16,), jnp.int32),
        pltpu.VMEM_SHARED((_NPAD,), jnp.float32),
        pltpu.VMEM_SHARED((_NPAD,), jnp.float32),
        pltpu.VMEM_SHARED((_NPAD,), jnp.float32),
        pltpu.VMEM_SHARED((_NPAD,), jnp.float32),
        pltpu.SemaphoreType.DMA,
        pltpu.SemaphoreType.DMA,
    ],
)
def _agg_sc(src_hbm, dst_hbm, y0_hbm, y1_hbm, iv_hbm, jv_hbm, z1_hbm,
            agg0_out, agg1_out, ci_out, cj_out,
            src_v, dst_v, yr0_v, yr1_v, mi_v, mj_v, iv_v, jv_v,
            agg0_sh, agg1_sh, ci_sh, cj_sh, sem0, sem1):
    c = lax.axis_index("c")
    s = lax.axis_index("s")
    w = _wid(c, s)
    sl = pl.ds(s * _NPT, _NPT)
    pltpu.sync_copy(z1_hbm.at[sl], agg0_sh.at[sl])
    pltpu.sync_copy(z1_hbm.at[sl], agg1_sh.at[sl])
    pltpu.sync_copy(z1_hbm.at[sl], ci_sh.at[sl])
    pltpu.sync_copy(z1_hbm.at[sl], cj_sh.at[sl])
    pltpu.sync_copy(iv_hbm, iv_v)
    pltpu.sync_copy(jv_hbm, jv_v)
    plsc.subcore_barrier()
    row0 = w * _RPT

    def outer(t, carry):
        rows = pl.ds(row0 + t * 16, 16)
        pltpu.sync_copy(src_hbm.at[rows], src_v)
        pltpu.sync_copy(dst_hbm.at[rows], dst_v)

        def inner(g, carry2):
            src_row = src_v.at[g]
            dst_row = dst_v.at[g]
            cp0 = pltpu.async_copy(y0_hbm.at[src_row], yr0_v, sem0)
            cp1 = pltpu.async_copy(y1_hbm.at[src_row], yr1_v, sem1)
            cp0.wait()
            cp1.wait()
            pltpu.sync_copy(yr0_v, agg0_sh.at[dst_row], add=True)
            pltpu.sync_copy(yr1_v, agg1_sh.at[dst_row], add=True)
            ivec = iv_v[...]
            jvec = jv_v[...]
            for k in range(8):
                d16 = dst_v[g, pl.ds(k * 16, 16)]
                mi_v[pl.ds(k * 16, 16)] = jnp.where(d16 == ivec, 1.0, 0.0)
                mj_v[pl.ds(k * 16, 16)] = jnp.where(d16 == jvec, 1.0, 0.0)
            pltpu.sync_copy(mi_v, ci_sh.at[src_row], add=True)
            pltpu.sync_copy(mj_v, cj_sh.at[src_row], add=True)
            return carry2

        lax.fori_loop(0, 16, inner, 0)
        return carry

    lax.fori_loop(0, _RPT // 16, outer, 0)
    plsc.subcore_barrier()
    pltpu.sync_copy(agg0_sh.at[sl], agg0_out.at[c, sl])
    pltpu.sync_copy(agg1_sh.at[sl], agg1_out.at[c, sl])
    pltpu.sync_copy(ci_sh.at[sl], ci_out.at[c, sl])
    pltpu.sync_copy(cj_sh.at[sl], cj_out.at[c, sl])


def _y_body(dp_ref, xt_ref, y0_ref, y1_ref):
    deg = dp_ref[0] + dp_ref[1]
    rs = lax.rsqrt(jnp.maximum(deg, 1.0))
    y0_ref[...] = xt_ref[0] * rs
    y1_ref[...] = xt_ref[1] * rs


_YB = 20480

_y_tc = pl.pallas_call(
    _y_body,
    grid=(_NPAD // _YB,),
    in_specs=[
        pl.BlockSpec((2, _YB), lambda k: (0, k)),
        pl.BlockSpec((2, _YB), lambda k: (0, k)),
    ],
    out_specs=[
        pl.BlockSpec((_YB,), lambda k: (k,)),
        pl.BlockSpec((_YB,), lambda k: (k,)),
    ],
    out_shape=[
        jax.ShapeDtypeStruct((_NPAD,), jnp.float32),
        jax.ShapeDtypeStruct((_NPAD,), jnp.float32),
    ],
)


def _h1_body(a0_ref, a1_ref, dp_ref, w0_ref, b0_ref, o_ref):
    a0 = a0_ref[0] + a0_ref[1]
    a1 = a1_ref[0] + a1_ref[1]
    deg = dp_ref[0] + dp_ref[1]
    rs = lax.rsqrt(jnp.maximum(deg, 1.0))
    a = jnp.stack([a0 * rs, a1 * rs], axis=1)
    h = jnp.dot(a, w0_ref[...], preferred_element_type=jnp.float32)
    h = jnp.maximum(h + b0_ref[...], 0.0)
    o_ref[...] = h * rs[:, None]


_h1_tc = pl.pallas_call(
    _h1_body,
    grid=(_NPAD // _B,),
    in_specs=[
        pl.BlockSpec((2, _B), lambda k: (0, k)),
        pl.BlockSpec((2, _B), lambda k: (0, k)),
        pl.BlockSpec((2, _B), lambda k: (0, k)),
        pl.BlockSpec((2, 16), lambda k: (0, 0)),
        pl.BlockSpec((1, 16), lambda k: (0, 0)),
    ],
    out_specs=pl.BlockSpec((_B, 16), lambda k: (k, 0)),
    out_shape=jax.ShapeDtypeStruct((_NPAD, 16), jnp.float32),
)


def _fin_body(ci_ref, cj_ref, h_ref, w1_ref, b1_ref, f1w_ref, f1b_ref,
              f2w_ref, f2b_ref, o_ref, acc, dsum):
    k = pl.program_id(0)

    @pl.when(k == 0)
    def _():
        acc[...] = jnp.zeros_like(acc)
        dsum[...] = jnp.zeros_like(dsum)

    ci = ci_ref[0] + ci_ref[1]
    cj = cj_ref[0] + cj_ref[1]
    c2 = jnp.stack([ci, cj])
    acc[...] += jnp.dot(c2, h_ref[...], preferred_element_type=jnp.float32)
    dsum[...] += jnp.sum(c2, axis=1).reshape(1, 2)

    @pl.when(k == pl.num_programs(0) - 1)
    def _():
        rs2 = lax.rsqrt(jnp.maximum(dsum[...], 1.0))
        agg2 = acc[...] * rs2.reshape(2, 1)
        h2 = jnp.dot(agg2, w1_ref[...], preferred_element_type=jnp.float32)
        h2 = jnp.maximum(h2 + b1_ref[...], 0.0)
        embd = jnp.concatenate([h2[0:1, :], h2[1:2, :]], axis=1)
        r = jnp.dot(embd, f1w_ref[...], preferred_element_type=jnp.float32)
        r = jnp.maximum(r + f1b_ref[...], 0.0)
        o_ref[...] = jnp.dot(r, f2w_ref[...],
                             preferred_element_type=jnp.float32) + f2b_ref[...]


_fin_tc = pl.pallas_call(
    _fin_body,
    grid=(_NPAD // _B,),
    in_specs=[
        pl.BlockSpec((2, _B), lambda k: (0, k)),
        pl.BlockSpec((2, _B), lambda k: (0, k)),
        pl.BlockSpec((_B, 16), lambda k: (k, 0)),
        pl.BlockSpec((16, 16), lambda k: (0, 0)),
        pl.BlockSpec((1, 16), lambda k: (0, 0)),
        pl.BlockSpec((32, 40), lambda k: (0, 0)),
        pl.BlockSpec((1, 40), lambda k: (0, 0)),
        pl.BlockSpec((40, 2), lambda k: (0, 0)),
        pl.BlockSpec((1, 2), lambda k: (0, 0)),
    ],
    out_specs=pl.BlockSpec((1, 2), lambda k: (0, 0)),
    out_shape=jax.ShapeDtypeStruct((1, 2), jnp.float32),
    scratch_shapes=[
        pltpu.VMEM((2, 16), jnp.float32),
        pltpu.VMEM((1, 2), jnp.float32),
    ],
)


def kernel(feature_torch, edge_torch, i, j, W0, b0, W1, b1, fc1_W, fc1_b,
           fc2_W, fc2_b):
    src = edge_torch[0]
    dst = edge_torch[1]
    pad = jnp.full((_EPAD - _E,), _DUMMY, jnp.int32)
    src2 = jnp.concatenate([src, pad]).reshape(_ROWS, 128)
    dst2 = jnp.concatenate([dst, pad]).reshape(_ROWS, 128)
    z1 = jnp.zeros((_NPAD,), jnp.float32)
    iv = jnp.full((16,), i, jnp.int32)
    jv = jnp.full((16,), j, jnp.int32)

    deg_part = _deg_sc(dst2, z1)
    xt = jnp.zeros((2, _NPAD), jnp.float32).at[:, :_N].set(feature_torch.T)
    y0, y1 = _y_tc(deg_part, xt)
    agg0_part, agg1_part, ci_part, cj_part = _agg_sc(
        src2, dst2, y0, y1, iv, jv, z1)
    h1s = _h1_tc(agg0_part, agg1_part, deg_part, W0, b0.reshape(1, 16))
    out = _fin_tc(ci_part, cj_part, h1s, W1, b1.reshape(1, 16),
                  fc1_W, fc1_b.reshape(1, 40), fc2_W, fc2_b.reshape(1, 2))
    return out.reshape(2)


# speedup vs baseline: 81.0021x; 1.1128x over previous
import functools

import jax
import jax.numpy as jnp
from jax import lax
from jax.experimental import pallas as pl
from jax.experimental.pallas import tpu as pltpu
from jax.experimental.pallas import tpu_sc as plsc

_N = 100000
_E = 3200000
_NPAD = 102400
_EPAD = 3276800
_ROWS = _EPAD // 128
_RPT = _ROWS // 32
_NPT = _NPAD // 16
_B = 12800
_DUMMY = _N

_mesh = plsc.VectorSubcoreMesh(core_axis_name="c", subcore_axis_name="s",
                               num_cores=2)


def _wid(c, s):
    return c * 16 + s


@functools.partial(
    pl.kernel,
    mesh=_mesh,
    out_type=jax.ShapeDtypeStruct((2, _NPAD), jnp.float32),
    scratch_types=[
        pltpu.VMEM((16, 128), jnp.int32),
        pltpu.VMEM((128,), jnp.float32),
        pltpu.VMEM_SHARED((_NPAD,), jnp.float32),
        pltpu.SemaphoreType.DMA,
    ],
)
def _deg_sc(dst_hbm, z1_hbm, deg_out, dst_v, ones_v, deg_sh, sem):
    c = lax.axis_index("c")
    s = lax.axis_index("s")
    w = _wid(c, s)
    for k in range(8):
        ones_v[pl.ds(k * 16, 16)] = jnp.full((16,), 1.0, jnp.float32)
    sl = pl.ds(s * _NPT, _NPT)
    pltpu.sync_copy(z1_hbm.at[sl], deg_sh.at[sl])
    plsc.subcore_barrier()
    row0 = w * _RPT

    def outer(t, carry):
        pltpu.sync_copy(dst_hbm.at[pl.ds(row0 + t * 16, 16)], dst_v)

        def inner(g, carry2):
            pltpu.sync_copy(ones_v, deg_sh.at[dst_v.at[g]], add=True)
            return carry2

        lax.fori_loop(0, 16, inner, 0)
        return carry

    lax.fori_loop(0, _RPT // 16, outer, 0)
    plsc.subcore_barrier()
    pltpu.sync_copy(deg_sh.at[sl], deg_out.at[c, sl])


@functools.partial(
    pl.kernel,
    mesh=_mesh,
    out_type=[
        jax.ShapeDtypeStruct((2, _NPAD), jnp.float32),
        jax.ShapeDtypeStruct((2, _NPAD), jnp.float32),
        jax.ShapeDtypeStruct((2, _NPAD), jnp.float32),
        jax.ShapeDtypeStruct((2, _NPAD), jnp.float32),
    ],
    scratch_types=[
        pltpu.VMEM((16, 128), jnp.int32),
        pltpu.VMEM((16, 128), jnp.int32),
        pltpu.VMEM((128,), jnp.float32),
        pltpu.VMEM((128,), jnp.float32),
        pltpu.VMEM((128,), jnp.float32),
        pltpu.VMEM((128,), jnp.float32),
        pltpu.VMEM((128,), jnp.float32),
        pltpu.VMEM((128,), jnp.float32),
        pltpu.VMEM((128,), jnp.float32),
        pltpu.VMEM((128,), jnp.float32),
        pltpu.VMEM((16,), jnp.int32),
        pltpu.VMEM((16,), jnp.int32),
        pltpu.VMEM_SHARED((_NPAD,), jnp.float32),
        pltpu.VMEM_SHARED((_NPAD,), jnp.float32),
        pltpu.VMEM_SHARED((_NPAD,), jnp.float32),
        pltpu.VMEM_SHARED((_NPAD,), jnp.float32),
        pltpu.SemaphoreType.DMA,
        pltpu.SemaphoreType.DMA,
    ],
)
def _agg_sc(src_hbm, dst_hbm, y0_hbm, y1_hbm, iv_hbm, jv_hbm, z1_hbm,
            agg0_out, agg1_out, ci_out, cj_out,
            src_v, dst_v, yr0_v, yr1_v, yb0_v, yb1_v,
            mi_v, mj_v, mb_i_v, mb_j_v, iv_v, jv_v,
            agg0_sh, agg1_sh, ci_sh, cj_sh, sem0, sem1):
    c = lax.axis_index("c")
    s = lax.axis_index("s")
    w = _wid(c, s)
    sl = pl.ds(s * _NPT, _NPT)
    pltpu.sync_copy(z1_hbm.at[sl], agg0_sh.at[sl])
    pltpu.sync_copy(z1_hbm.at[sl], agg1_sh.at[sl])
    pltpu.sync_copy(z1_hbm.at[sl], ci_sh.at[sl])
    pltpu.sync_copy(z1_hbm.at[sl], cj_sh.at[sl])
    pltpu.sync_copy(iv_hbm, iv_v)
    pltpu.sync_copy(jv_hbm, jv_v)
    plsc.subcore_barrier()
    row0 = w * _RPT

    def outer(t, carry):
        rows = pl.ds(row0 + t * 16, 16)
        pltpu.sync_copy(src_hbm.at[rows], src_v)
        pltpu.sync_copy(dst_hbm.at[rows], dst_v)
        ivec = iv_v[...]
        jvec = jv_v[...]

        def masks(g, mi_v, mj_v):
            for k in range(8):
                d16 = dst_v[g, pl.ds(k * 16, 16)]
                mi_v[pl.ds(k * 16, 16)] = jnp.where(d16 == ivec, 1.0, 0.0)
                mj_v[pl.ds(k * 16, 16)] = jnp.where(d16 == jvec, 1.0, 0.0)

        def inner(p, carry2):
            g0 = 2 * p
            g1 = 2 * p + 1
            cpa0 = pltpu.async_copy(y0_hbm.at[src_v.at[g0]], yr0_v, sem0)
            cpa1 = pltpu.async_copy(y1_hbm.at[src_v.at[g0]], yr1_v, sem1)
            masks(g0, mi_v, mj_v)
            cpb0 = pltpu.async_copy(y0_hbm.at[src_v.at[g1]], yb0_v, sem0)
            cpb1 = pltpu.async_copy(y1_hbm.at[src_v.at[g1]], yb1_v, sem1)
            cpa0.wait()
            cpa1.wait()
            pltpu.sync_copy(yr0_v, agg0_sh.at[dst_v.at[g0]], add=True)
            pltpu.sync_copy(yr1_v, agg1_sh.at[dst_v.at[g0]], add=True)
            pltpu.sync_copy(mi_v, ci_sh.at[src_v.at[g0]], add=True)
            pltpu.sync_copy(mj_v, cj_sh.at[src_v.at[g0]], add=True)
            masks(g1, mb_i_v, mb_j_v)
            cpb0.wait()
            cpb1.wait()
            pltpu.sync_copy(yb0_v, agg0_sh.at[dst_v.at[g1]], add=True)
            pltpu.sync_copy(yb1_v, agg1_sh.at[dst_v.at[g1]], add=True)
            pltpu.sync_copy(mb_i_v, ci_sh.at[src_v.at[g1]], add=True)
            pltpu.sync_copy(mb_j_v, cj_sh.at[src_v.at[g1]], add=True)
            return carry2

        lax.fori_loop(0, 8, inner, 0)
        return carry

    lax.fori_loop(0, _RPT // 16, outer, 0)
    plsc.subcore_barrier()
    pltpu.sync_copy(agg0_sh.at[sl], agg0_out.at[c, sl])
    pltpu.sync_copy(agg1_sh.at[sl], agg1_out.at[c, sl])
    pltpu.sync_copy(ci_sh.at[sl], ci_out.at[c, sl])
    pltpu.sync_copy(cj_sh.at[sl], cj_out.at[c, sl])


def _y_body(dp_ref, xt_ref, y0_ref, y1_ref):
    deg = dp_ref[0] + dp_ref[1]
    rs = lax.rsqrt(jnp.maximum(deg, 1.0))
    y0_ref[...] = xt_ref[0] * rs
    y1_ref[...] = xt_ref[1] * rs


_YB = 20480

_y_tc = pl.pallas_call(
    _y_body,
    grid=(_NPAD // _YB,),
    in_specs=[
        pl.BlockSpec((2, _YB), lambda k: (0, k)),
        pl.BlockSpec((2, _YB), lambda k: (0, k)),
    ],
    out_specs=[
        pl.BlockSpec((_YB,), lambda k: (k,)),
        pl.BlockSpec((_YB,), lambda k: (k,)),
    ],
    out_shape=[
        jax.ShapeDtypeStruct((_NPAD,), jnp.float32),
        jax.ShapeDtypeStruct((_NPAD,), jnp.float32),
    ],
)


def _h1_body(a0_ref, a1_ref, dp_ref, w0_ref, b0_ref, o_ref):
    a0 = a0_ref[0] + a0_ref[1]
    a1 = a1_ref[0] + a1_ref[1]
    deg = dp_ref[0] + dp_ref[1]
    rs = lax.rsqrt(jnp.maximum(deg, 1.0))
    a = jnp.stack([a0 * rs, a1 * rs], axis=1)
    h = jnp.dot(a, w0_ref[...], preferred_element_type=jnp.float32)
    h = jnp.maximum(h + b0_ref[...], 0.0)
    o_ref[...] = h * rs[:, None]


_h1_tc = pl.pallas_call(
    _h1_body,
    grid=(_NPAD // _B,),
    in_specs=[
        pl.BlockSpec((2, _B), lambda k: (0, k)),
        pl.BlockSpec((2, _B), lambda k: (0, k)),
        pl.BlockSpec((2, _B), lambda k: (0, k)),
        pl.BlockSpec((2, 16), lambda k: (0, 0)),
        pl.BlockSpec((1, 16), lambda k: (0, 0)),
    ],
    out_specs=pl.BlockSpec((_B, 16), lambda k: (k, 0)),
    out_shape=jax.ShapeDtypeStruct((_NPAD, 16), jnp.float32),
)


def _fin_body(ci_ref, cj_ref, h_ref, w1_ref, b1_ref, f1w_ref, f1b_ref,
              f2w_ref, f2b_ref, o_ref, acc, dsum):
    k = pl.program_id(0)

    @pl.when(k == 0)
    def _():
        acc[...] = jnp.zeros_like(acc)
        dsum[...] = jnp.zeros_like(dsum)

    ci = ci_ref[0] + ci_ref[1]
    cj = cj_ref[0] + cj_ref[1]
    c2 = jnp.stack([ci, cj])
    acc[...] += jnp.dot(c2, h_ref[...], preferred_element_type=jnp.float32)
    dsum[...] += jnp.sum(c2, axis=1).reshape(1, 2)

    @pl.when(k == pl.num_programs(0) - 1)
    def _():
        rs2 = lax.rsqrt(jnp.maximum(dsum[...], 1.0))
        agg2 = acc[...] * rs2.reshape(2, 1)
        h2 = jnp.dot(agg2, w1_ref[...], preferred_element_type=jnp.float32)
        h2 = jnp.maximum(h2 + b1_ref[...], 0.0)
        embd = jnp.concatenate([h2[0:1, :], h2[1:2, :]], axis=1)
        r = jnp.dot(embd, f1w_ref[...], preferred_element_type=jnp.float32)
        r = jnp.maximum(r + f1b_ref[...], 0.0)
        o_ref[...] = jnp.dot(r, f2w_ref[...],
                             preferred_element_type=jnp.float32) + f2b_ref[...]


_fin_tc = pl.pallas_call(
    _fin_body,
    grid=(_NPAD // _B,),
    in_specs=[
        pl.BlockSpec((2, _B), lambda k: (0, k)),
        pl.BlockSpec((2, _B), lambda k: (0, k)),
        pl.BlockSpec((_B, 16), lambda k: (k, 0)),
        pl.BlockSpec((16, 16), lambda k: (0, 0)),
        pl.BlockSpec((1, 16), lambda k: (0, 0)),
        pl.BlockSpec((32, 40), lambda k: (0, 0)),
        pl.BlockSpec((1, 40), lambda k: (0, 0)),
        pl.BlockSpec((40, 2), lambda k: (0, 0)),
        pl.BlockSpec((1, 2), lambda k: (0, 0)),
    ],
    out_specs=pl.BlockSpec((1, 2), lambda k: (0, 0)),
    out_shape=jax.ShapeDtypeStruct((1, 2), jnp.float32),
    scratch_shapes=[
        pltpu.VMEM((2, 16), jnp.float32),
        pltpu.VMEM((1, 2), jnp.float32),
    ],
)


def kernel(feature_torch, edge_torch, i, j, W0, b0, W1, b1, fc1_W, fc1_b,
           fc2_W, fc2_b):
    src = edge_torch[0]
    dst = edge_torch[1]
    pad = jnp.full((_EPAD - _E,), _DUMMY, jnp.int32)
    src2 = jnp.concatenate([src, pad]).reshape(_ROWS, 128)
    dst2 = jnp.concatenate([dst, pad]).reshape(_ROWS, 128)
    z1 = jnp.zeros((_NPAD,), jnp.float32)
    iv = jnp.full((16,), i, jnp.int32)
    jv = jnp.full((16,), j, jnp.int32)

    deg_part = _deg_sc(dst2, z1)
    xt = jnp.zeros((2, _NPAD), jnp.float32).at[:, :_N].set(feature_torch.T)
    y0, y1 = _y_tc(deg_part, xt)
    agg0_part, agg1_part, ci_part, cj_part = _agg_sc(
        src2, dst2, y0, y1, iv, jv, z1)
    h1s = _h1_tc(agg0_part, agg1_part, deg_part, W0, b0.reshape(1, 16))
    out = _fin_tc(ci_part, cj_part, h1s, W1, b1.reshape(1, 16),
                  fc1_W, fc1_b.reshape(1, 40), fc2_W, fc2_b.reshape(1, 2))
    return out.reshape(2)


# speedup vs baseline: 129.2983x; 1.5962x over previous
import functools

import jax
import jax.numpy as jnp
from jax import lax
from jax.experimental import pallas as pl
from jax.experimental.pallas import tpu as pltpu
from jax.experimental.pallas import tpu_sc as plsc

_N = 100000
_E = 3200000
_NPAD = 102400
_EPAD = 3276800
_ROWS = _EPAD // 128
_RPT = _ROWS // 32
_NPT = _NPAD // 16
_B = 12800
_DUMMY = _N

_mesh = plsc.VectorSubcoreMesh(core_axis_name="c", subcore_axis_name="s",
                               num_cores=2)


def _wid(c, s):
    return c * 16 + s


@functools.partial(
    pl.kernel,
    mesh=_mesh,
    out_type=jax.ShapeDtypeStruct((2, _NPAD), jnp.float32),
    scratch_types=[
        pltpu.VMEM((16, 128), jnp.int32),
        pltpu.VMEM((128,), jnp.float32),
        pltpu.VMEM_SHARED((_NPAD,), jnp.float32),
        pltpu.SemaphoreType.DMA,
    ],
)
def _deg_sc(dst_hbm, z1_hbm, deg_out, dst_v, ones_v, deg_sh, sem):
    c = lax.axis_index("c")
    s = lax.axis_index("s")
    w = _wid(c, s)
    for k in range(8):
        ones_v[pl.ds(k * 16, 16)] = jnp.full((16,), 1.0, jnp.float32)
    sl = pl.ds(s * _NPT, _NPT)
    pltpu.sync_copy(z1_hbm.at[sl], deg_sh.at[sl])
    plsc.subcore_barrier()
    row0 = w * _RPT

    def outer(t, carry):
        pltpu.sync_copy(dst_hbm.at[pl.ds(row0 + t * 16, 16)], dst_v)

        def inner(g, carry2):
            pltpu.sync_copy(ones_v, deg_sh.at[dst_v.at[g]], add=True)
            return carry2

        lax.fori_loop(0, 16, inner, 0)
        return carry

    lax.fori_loop(0, _RPT // 16, outer, 0)
    plsc.subcore_barrier()
    pltpu.sync_copy(deg_sh.at[sl], deg_out.at[c, sl])


@functools.partial(
    pl.kernel,
    mesh=_mesh,
    out_type=[
        jax.ShapeDtypeStruct((2, _NPAD), jnp.float32),
        jax.ShapeDtypeStruct((2, _NPAD), jnp.float32),
        jax.ShapeDtypeStruct((2, _NPAD), jnp.float32),
        jax.ShapeDtypeStruct((2, _NPAD), jnp.float32),
    ],
    scratch_types=[
        pltpu.VMEM((16, 128), jnp.int32),
        pltpu.VMEM((16, 128), jnp.int32),
        pltpu.VMEM((128,), jnp.float32),
        pltpu.VMEM((128,), jnp.float32),
        pltpu.VMEM((128,), jnp.float32),
        pltpu.VMEM((128,), jnp.float32),
        pltpu.VMEM((128,), jnp.float32),
        pltpu.VMEM((128,), jnp.float32),
        pltpu.VMEM((128,), jnp.float32),
        pltpu.VMEM((128,), jnp.float32),
        pltpu.VMEM((16,), jnp.int32),
        pltpu.VMEM((16,), jnp.int32),
        pltpu.VMEM_SHARED((_NPAD,), jnp.float32),
        pltpu.VMEM_SHARED((_NPAD,), jnp.float32),
        pltpu.VMEM_SHARED((_NPAD,), jnp.float32),
        pltpu.VMEM_SHARED((_NPAD,), jnp.float32),
        pltpu.VMEM_SHARED((_NPAD,), jnp.float32),
        pltpu.VMEM_SHARED((_NPAD,), jnp.float32),
        pltpu.SemaphoreType.DMA,
        pltpu.SemaphoreType.DMA,
        pltpu.SemaphoreType.DMA,
        pltpu.SemaphoreType.DMA,
    ],
)
def _agg_sc(src_hbm, dst_hbm, y0_hbm, y1_hbm, iv_hbm, jv_hbm, z1_hbm,
            agg0_out, agg1_out, ci_out, cj_out,
            src_v, dst_v, yr0_v, yr1_v, yb0_v, yb1_v,
            mi_v, mj_v, mb_i_v, mb_j_v, iv_v, jv_v,
            y0_sh, y1_sh, agg0_sh, agg1_sh, ci_sh, cj_sh,
            sem0, sem1, sg0, sg1):
    c = lax.axis_index("c")
    s = lax.axis_index("s")
    w = _wid(c, s)
    sl = pl.ds(s * _NPT, _NPT)
    pltpu.sync_copy(y0_hbm.at[sl], y0_sh.at[sl])
    pltpu.sync_copy(y1_hbm.at[sl], y1_sh.at[sl])
    pltpu.sync_copy(z1_hbm.at[sl], agg0_sh.at[sl])
    pltpu.sync_copy(z1_hbm.at[sl], agg1_sh.at[sl])
    pltpu.sync_copy(z1_hbm.at[sl], ci_sh.at[sl])
    pltpu.sync_copy(z1_hbm.at[sl], cj_sh.at[sl])
    pltpu.sync_copy(iv_hbm, iv_v)
    pltpu.sync_copy(jv_hbm, jv_v)
    plsc.subcore_barrier()
    row0 = w * _RPT

    def outer(t, carry):
        rows = pl.ds(row0 + t * 16, 16)
        pltpu.sync_copy(src_hbm.at[rows], src_v)
        pltpu.sync_copy(dst_hbm.at[rows], dst_v)
        ivec = iv_v[...]
        jvec = jv_v[...]

        def masks(g, mi_v, mj_v):
            for k in range(8):
                d16 = dst_v[g, pl.ds(k * 16, 16)]
                mi_v[pl.ds(k * 16, 16)] = jnp.where(d16 == ivec, 1.0, 0.0)
                mj_v[pl.ds(k * 16, 16)] = jnp.where(d16 == jvec, 1.0, 0.0)

        def inner(p, carry2):
            g0 = 2 * p
            g1 = 2 * p + 1
            cpa0 = pltpu.async_copy(y0_sh.at[src_v.at[g0]], yr0_v, sem0)
            cpa1 = pltpu.async_copy(y1_sh.at[src_v.at[g0]], yr1_v, sem1)
            masks(g0, mi_v, mj_v)
            cpb0 = pltpu.async_copy(y0_sh.at[src_v.at[g1]], yb0_v, sem0)
            cpb1 = pltpu.async_copy(y1_sh.at[src_v.at[g1]], yb1_v, sem1)
            cpa0.wait()
            cpa1.wait()
            sa0 = pltpu.async_copy(yr0_v, agg0_sh.at[dst_v.at[g0]], sg0,
                                   add=True)
            sa1 = pltpu.async_copy(yr1_v, agg1_sh.at[dst_v.at[g0]], sg1,
                                   add=True)
            sa2 = pltpu.async_copy(mi_v, ci_sh.at[src_v.at[g0]], sg0,
                                   add=True)
            sa3 = pltpu.async_copy(mj_v, cj_sh.at[src_v.at[g0]], sg1,
                                   add=True)
            masks(g1, mb_i_v, mb_j_v)
            cpb0.wait()
            cpb1.wait()
            sa0.wait()
            sa1.wait()
            sa2.wait()
            sa3.wait()
            sb0 = pltpu.async_copy(yb0_v, agg0_sh.at[dst_v.at[g1]], sg0,
                                   add=True)
            sb1 = pltpu.async_copy(yb1_v, agg1_sh.at[dst_v.at[g1]], sg1,
                                   add=True)
            sb2 = pltpu.async_copy(mb_i_v, ci_sh.at[src_v.at[g1]], sg0,
                                   add=True)
            sb3 = pltpu.async_copy(mb_j_v, cj_sh.at[src_v.at[g1]], sg1,
                                   add=True)
            sb0.wait()
            sb1.wait()
            sb2.wait()
            sb3.wait()
            return carry2

        lax.fori_loop(0, 8, inner, 0)
        return carry

    lax.fori_loop(0, _RPT // 16, outer, 0)
    plsc.subcore_barrier()
    pltpu.sync_copy(agg0_sh.at[sl], agg0_out.at[c, sl])
    pltpu.sync_copy(agg1_sh.at[sl], agg1_out.at[c, sl])
    pltpu.sync_copy(ci_sh.at[sl], ci_out.at[c, sl])
    pltpu.sync_copy(cj_sh.at[sl], cj_out.at[c, sl])


def _y_body(dp_ref, xt_ref, y0_ref, y1_ref):
    deg = dp_ref[0] + dp_ref[1]
    rs = lax.rsqrt(jnp.maximum(deg, 1.0))
    y0_ref[...] = xt_ref[0] * rs
    y1_ref[...] = xt_ref[1] * rs


_YB = 20480

_y_tc = pl.pallas_call(
    _y_body,
    grid=(_NPAD // _YB,),
    in_specs=[
        pl.BlockSpec((2, _YB), lambda k: (0, k)),
        pl.BlockSpec((2, _YB), lambda k: (0, k)),
    ],
    out_specs=[
        pl.BlockSpec((_YB,), lambda k: (k,)),
        pl.BlockSpec((_YB,), lambda k: (k,)),
    ],
    out_shape=[
        jax.ShapeDtypeStruct((_NPAD,), jnp.float32),
        jax.ShapeDtypeStruct((_NPAD,), jnp.float32),
    ],
)


def _h1_body(a0_ref, a1_ref, dp_ref, w0_ref, b0_ref, o_ref):
    a0 = a0_ref[0] + a0_ref[1]
    a1 = a1_ref[0] + a1_ref[1]
    deg = dp_ref[0] + dp_ref[1]
    rs = lax.rsqrt(jnp.maximum(deg, 1.0))
    a = jnp.stack([a0 * rs, a1 * rs], axis=1)
    h = jnp.dot(a, w0_ref[...], preferred_element_type=jnp.float32)
    h = jnp.maximum(h + b0_ref[...], 0.0)
    o_ref[...] = h * rs[:, None]


_h1_tc = pl.pallas_call(
    _h1_body,
    grid=(_NPAD // _B,),
    in_specs=[
        pl.BlockSpec((2, _B), lambda k: (0, k)),
        pl.BlockSpec((2, _B), lambda k: (0, k)),
        pl.BlockSpec((2, _B), lambda k: (0, k)),
        pl.BlockSpec((2, 16), lambda k: (0, 0)),
        pl.BlockSpec((1, 16), lambda k: (0, 0)),
    ],
    out_specs=pl.BlockSpec((_B, 16), lambda k: (k, 0)),
    out_shape=jax.ShapeDtypeStruct((_NPAD, 16), jnp.float32),
)


def _fin_body(ci_ref, cj_ref, h_ref, w1_ref, b1_ref, f1w_ref, f1b_ref,
              f2w_ref, f2b_ref, o_ref, acc, dsum):
    k = pl.program_id(0)

    @pl.when(k == 0)
    def _():
        acc[...] = jnp.zeros_like(acc)
        dsum[...] = jnp.zeros_like(dsum)

    ci = ci_ref[0] + ci_ref[1]
    cj = cj_ref[0] + cj_ref[1]
    h = h_ref[...]
    acc[0:1, :] += jnp.sum(ci[:, None] * h, axis=0, keepdims=True)
    acc[1:2, :] += jnp.sum(cj[:, None] * h, axis=0, keepdims=True)
    dsum[...] += jnp.stack([jnp.sum(ci), jnp.sum(cj)]).reshape(1, 2)

    @pl.when(k == pl.num_programs(0) - 1)
    def _():
        rs2 = lax.rsqrt(jnp.maximum(dsum[...], 1.0))
        agg2 = acc[...] * rs2.reshape(2, 1)
        h2 = jnp.dot(agg2, w1_ref[...], preferred_element_type=jnp.float32)
        h2 = jnp.maximum(h2 + b1_ref[...], 0.0)
        embd = jnp.concatenate([h2[0:1, :], h2[1:2, :]], axis=1)
        r = jnp.dot(embd, f1w_ref[...], preferred_element_type=jnp.float32)
        r = jnp.maximum(r + f1b_ref[...], 0.0)
        o_ref[...] = jnp.dot(r, f2w_ref[...],
                             preferred_element_type=jnp.float32) + f2b_ref[...]


_fin_tc = pl.pallas_call(
    _fin_body,
    grid=(_NPAD // _B,),
    in_specs=[
        pl.BlockSpec((2, _B), lambda k: (0, k)),
        pl.BlockSpec((2, _B), lambda k: (0, k)),
        pl.BlockSpec((_B, 16), lambda k: (k, 0)),
        pl.BlockSpec((16, 16), lambda k: (0, 0)),
        pl.BlockSpec((1, 16), lambda k: (0, 0)),
        pl.BlockSpec((32, 40), lambda k: (0, 0)),
        pl.BlockSpec((1, 40), lambda k: (0, 0)),
        pl.BlockSpec((40, 2), lambda k: (0, 0)),
        pl.BlockSpec((1, 2), lambda k: (0, 0)),
    ],
    out_specs=pl.BlockSpec((1, 2), lambda k: (0, 0)),
    out_shape=jax.ShapeDtypeStruct((1, 2), jnp.float32),
    scratch_shapes=[
        pltpu.VMEM((2, 16), jnp.float32),
        pltpu.VMEM((1, 2), jnp.float32),
    ],
)


def kernel(feature_torch, edge_torch, i, j, W0, b0, W1, b1, fc1_W, fc1_b,
           fc2_W, fc2_b):
    src = edge_torch[0]
    dst = edge_torch[1]
    pad = jnp.full((_EPAD - _E,), _DUMMY, jnp.int32)
    src2 = jnp.concatenate([src, pad]).reshape(_ROWS, 128)
    dst2 = jnp.concatenate([dst, pad]).reshape(_ROWS, 128)
    z1 = jnp.zeros((_NPAD,), jnp.float32)
    iv = jnp.full((16,), i, jnp.int32)
    jv = jnp.full((16,), j, jnp.int32)

    deg_part = _deg_sc(dst2, z1)
    xt = jnp.zeros((2, _NPAD), jnp.float32).at[:, :_N].set(feature_torch.T)
    y0, y1 = _y_tc(deg_part, xt)
    agg0_part, agg1_part, ci_part, cj_part = _agg_sc(
        src2, dst2, y0, y1, iv, jv, z1)
    h1s = _h1_tc(agg0_part, agg1_part, deg_part, W0, b0.reshape(1, 16))
    out = _fin_tc(ci_part, cj_part, h1s, W1, b1.reshape(1, 16),
                  fc1_W, fc1_b.reshape(1, 40), fc2_W, fc2_b.reshape(1, 2))
    return out.reshape(2)


# speedup vs baseline: 130.6400x; 1.0104x over previous
import functools

import jax
import jax.numpy as jnp
from jax import lax
from jax.experimental import pallas as pl
from jax.experimental.pallas import tpu as pltpu
from jax.experimental.pallas import tpu_sc as plsc

_N = 100000
_E = 3200000
_NPAD = 102400
_EPAD = 3276800
_ROWS = _EPAD // 128
_RPT = _ROWS // 32
_NPT = _NPAD // 16
_B = 12800
_DUMMY = _N

_mesh = plsc.VectorSubcoreMesh(core_axis_name="c", subcore_axis_name="s",
                               num_cores=2)


def _wid(c, s):
    return c * 16 + s


@functools.partial(
    pl.kernel,
    mesh=_mesh,
    out_type=jax.ShapeDtypeStruct((2, _NPAD), jnp.float32),
    scratch_types=[
        pltpu.VMEM((16, 128), jnp.int32),
        pltpu.VMEM((128,), jnp.float32),
        pltpu.VMEM_SHARED((_NPAD,), jnp.float32),
        pltpu.SemaphoreType.DMA,
        pltpu.SemaphoreType.DMA,
    ],
)
def _deg_sc(dst_hbm, z1_hbm, deg_out, dst_v, ones_v, deg_sh, sem, sem1):
    c = lax.axis_index("c")
    s = lax.axis_index("s")
    w = _wid(c, s)
    for k in range(8):
        ones_v[pl.ds(k * 16, 16)] = jnp.full((16,), 1.0, jnp.float32)
    sl = pl.ds(s * _NPT, _NPT)
    pltpu.sync_copy(z1_hbm.at[sl], deg_sh.at[sl])
    plsc.subcore_barrier()
    row0 = w * _RPT

    def outer(t, carry):
        pltpu.sync_copy(dst_hbm.at[pl.ds(row0 + t * 16, 16)], dst_v)

        def inner(p, carry2):
            g0 = 4 * p
            s0 = pltpu.async_copy(ones_v, deg_sh.at[dst_v.at[g0]], sem,
                                  add=True)
            s1 = pltpu.async_copy(ones_v, deg_sh.at[dst_v.at[g0 + 1]], sem1,
                                  add=True)
            s2 = pltpu.async_copy(ones_v, deg_sh.at[dst_v.at[g0 + 2]], sem,
                                  add=True)
            s3 = pltpu.async_copy(ones_v, deg_sh.at[dst_v.at[g0 + 3]], sem1,
                                  add=True)
            s0.wait()
            s1.wait()
            s2.wait()
            s3.wait()
            return carry2

        lax.fori_loop(0, 4, inner, 0)
        return carry

    lax.fori_loop(0, _RPT // 16, outer, 0)
    plsc.subcore_barrier()
    pltpu.sync_copy(deg_sh.at[sl], deg_out.at[c, sl])


@functools.partial(
    pl.kernel,
    mesh=_mesh,
    out_type=[
        jax.ShapeDtypeStruct((2, _NPAD), jnp.float32),
        jax.ShapeDtypeStruct((2, _NPAD), jnp.float32),
        jax.ShapeDtypeStruct((2, _NPAD), jnp.float32),
        jax.ShapeDtypeStruct((2, _NPAD), jnp.float32),
    ],
    scratch_types=[
        pltpu.VMEM((16, 128), jnp.int32),
        pltpu.VMEM((16, 128), jnp.int32),
        pltpu.VMEM((128,), jnp.float32),
        pltpu.VMEM((128,), jnp.float32),
        pltpu.VMEM((128,), jnp.float32),
        pltpu.VMEM((128,), jnp.float32),
        pltpu.VMEM((128,), jnp.float32),
        pltpu.VMEM((128,), jnp.float32),
        pltpu.VMEM((128,), jnp.float32),
        pltpu.VMEM((128,), jnp.float32),
        pltpu.VMEM((16,), jnp.int32),
        pltpu.VMEM((16,), jnp.int32),
        pltpu.VMEM_SHARED((_NPAD,), jnp.float32),
        pltpu.VMEM_SHARED((_NPAD,), jnp.float32),
        pltpu.VMEM_SHARED((_NPAD,), jnp.float32),
        pltpu.VMEM_SHARED((_NPAD,), jnp.float32),
        pltpu.VMEM_SHARED((_NPAD,), jnp.float32),
        pltpu.VMEM_SHARED((_NPAD,), jnp.float32),
        pltpu.SemaphoreType.DMA,
        pltpu.SemaphoreType.DMA,
        pltpu.SemaphoreType.DMA,
        pltpu.SemaphoreType.DMA,
    ],
)
def _agg_sc(src_hbm, dst_hbm, y0_hbm, y1_hbm, iv_hbm, jv_hbm, z1_hbm,
            agg0_out, agg1_out, ci_out, cj_out,
            src_v, dst_v, yr0_v, yr1_v, yb0_v, yb1_v,
            mi_v, mj_v, mb_i_v, mb_j_v, iv_v, jv_v,
            y0_sh, y1_sh, agg0_sh, agg1_sh, ci_sh, cj_sh,
            sem0, sem1, sg0, sg1):
    c = lax.axis_index("c")
    s = lax.axis_index("s")
    w = _wid(c, s)
    sl = pl.ds(s * _NPT, _NPT)
    pltpu.sync_copy(y0_hbm.at[sl], y0_sh.at[sl])
    pltpu.sync_copy(y1_hbm.at[sl], y1_sh.at[sl])
    pltpu.sync_copy(z1_hbm.at[sl], agg0_sh.at[sl])
    pltpu.sync_copy(z1_hbm.at[sl], agg1_sh.at[sl])
    pltpu.sync_copy(z1_hbm.at[sl], ci_sh.at[sl])
    pltpu.sync_copy(z1_hbm.at[sl], cj_sh.at[sl])
    pltpu.sync_copy(iv_hbm, iv_v)
    pltpu.sync_copy(jv_hbm, jv_v)
    plsc.subcore_barrier()
    row0 = w * _RPT

    def outer(t, carry):
        rows = pl.ds(row0 + t * 16, 16)
        pltpu.sync_copy(src_hbm.at[rows], src_v)
        pltpu.sync_copy(dst_hbm.at[rows], dst_v)
        ivec = iv_v[...]
        jvec = jv_v[...]

        def masks(g, mi_v, mj_v):
            for k in range(8):
                d16 = dst_v[g, pl.ds(k * 16, 16)]
                mi_v[pl.ds(k * 16, 16)] = jnp.where(d16 == ivec, 1.0, 0.0)
                mj_v[pl.ds(k * 16, 16)] = jnp.where(d16 == jvec, 1.0, 0.0)

        def inner(p, carry2):
            g0 = 2 * p
            g1 = 2 * p + 1
            cpa0 = pltpu.async_copy(y0_sh.at[src_v.at[g0]], yr0_v, sem0)
            cpa1 = pltpu.async_copy(y1_sh.at[src_v.at[g0]], yr1_v, sem1)
            masks(g0, mi_v, mj_v)
            cpb0 = pltpu.async_copy(y0_sh.at[src_v.at[g1]], yb0_v, sem0)
            cpb1 = pltpu.async_copy(y1_sh.at[src_v.at[g1]], yb1_v, sem1)
            cpa0.wait()
            cpa1.wait()
            sa0 = pltpu.async_copy(yr0_v, agg0_sh.at[dst_v.at[g0]], sg0,
                                   add=True)
            sa1 = pltpu.async_copy(yr1_v, agg1_sh.at[dst_v.at[g0]], sg1,
                                   add=True)
            sa2 = pltpu.async_copy(mi_v, ci_sh.at[src_v.at[g0]], sg0,
                                   add=True)
            sa3 = pltpu.async_copy(mj_v, cj_sh.at[src_v.at[g0]], sg1,
                                   add=True)
            masks(g1, mb_i_v, mb_j_v)
            cpb0.wait()
            cpb1.wait()
            sa0.wait()
            sa1.wait()
            sa2.wait()
            sa3.wait()
            sb0 = pltpu.async_copy(yb0_v, agg0_sh.at[dst_v.at[g1]], sg0,
                                   add=True)
            sb1 = pltpu.async_copy(yb1_v, agg1_sh.at[dst_v.at[g1]], sg1,
                                   add=True)
            sb2 = pltpu.async_copy(mb_i_v, ci_sh.at[src_v.at[g1]], sg0,
                                   add=True)
            sb3 = pltpu.async_copy(mb_j_v, cj_sh.at[src_v.at[g1]], sg1,
                                   add=True)
            sb0.wait()
            sb1.wait()
            sb2.wait()
            sb3.wait()
            return carry2

        lax.fori_loop(0, 8, inner, 0)
        return carry

    lax.fori_loop(0, _RPT // 16, outer, 0)
    plsc.subcore_barrier()
    pltpu.sync_copy(agg0_sh.at[sl], agg0_out.at[c, sl])
    pltpu.sync_copy(agg1_sh.at[sl], agg1_out.at[c, sl])
    pltpu.sync_copy(ci_sh.at[sl], ci_out.at[c, sl])
    pltpu.sync_copy(cj_sh.at[sl], cj_out.at[c, sl])


def _y_body(dp_ref, xt_ref, y0_ref, y1_ref):
    deg = dp_ref[0] + dp_ref[1]
    rs = lax.rsqrt(jnp.maximum(deg, 1.0))
    y0_ref[...] = xt_ref[0] * rs
    y1_ref[...] = xt_ref[1] * rs


_YB = 20480

_y_tc = pl.pallas_call(
    _y_body,
    grid=(_NPAD // _YB,),
    in_specs=[
        pl.BlockSpec((2, _YB), lambda k: (0, k)),
        pl.BlockSpec((2, _YB), lambda k: (0, k)),
    ],
    out_specs=[
        pl.BlockSpec((_YB,), lambda k: (k,)),
        pl.BlockSpec((_YB,), lambda k: (k,)),
    ],
    out_shape=[
        jax.ShapeDtypeStruct((_NPAD,), jnp.float32),
        jax.ShapeDtypeStruct((_NPAD,), jnp.float32),
    ],
)


def _h1_body(a0_ref, a1_ref, dp_ref, w0_ref, b0_ref, o_ref):
    a0 = a0_ref[0] + a0_ref[1]
    a1 = a1_ref[0] + a1_ref[1]
    deg = dp_ref[0] + dp_ref[1]
    rs = lax.rsqrt(jnp.maximum(deg, 1.0))
    a = jnp.stack([a0 * rs, a1 * rs], axis=1)
    h = jnp.dot(a, w0_ref[...], preferred_element_type=jnp.float32)
    h = jnp.maximum(h + b0_ref[...], 0.0)
    o_ref[...] = h * rs[:, None]


_h1_tc = pl.pallas_call(
    _h1_body,
    grid=(_NPAD // _B,),
    in_specs=[
        pl.BlockSpec((2, _B), lambda k: (0, k)),
        pl.BlockSpec((2, _B), lambda k: (0, k)),
        pl.BlockSpec((2, _B), lambda k: (0, k)),
        pl.BlockSpec((2, 16), lambda k: (0, 0)),
        pl.BlockSpec((1, 16), lambda k: (0, 0)),
    ],
    out_specs=pl.BlockSpec((_B, 16), lambda k: (k, 0)),
    out_shape=jax.ShapeDtypeStruct((_NPAD, 16), jnp.float32),
)


def _fin_body(ci_ref, cj_ref, h_ref, w1_ref, b1_ref, f1w_ref, f1b_ref,
              f2w_ref, f2b_ref, o_ref, acc, dsum):
    k = pl.program_id(0)

    @pl.when(k == 0)
    def _():
        acc[...] = jnp.zeros_like(acc)
        dsum[...] = jnp.zeros_like(dsum)

    ci = ci_ref[0] + ci_ref[1]
    cj = cj_ref[0] + cj_ref[1]
    h = h_ref[...]
    acc[0:1, :] += jnp.sum(ci[:, None] * h, axis=0, keepdims=True)
    acc[1:2, :] += jnp.sum(cj[:, None] * h, axis=0, keepdims=True)
    dsum[...] += jnp.stack([jnp.sum(ci), jnp.sum(cj)]).reshape(1, 2)

    @pl.when(k == pl.num_programs(0) - 1)
    def _():
        rs2 = lax.rsqrt(jnp.maximum(dsum[...], 1.0))
        agg2 = acc[...] * rs2.reshape(2, 1)
        h2 = jnp.dot(agg2, w1_ref[...], preferred_element_type=jnp.float32)
        h2 = jnp.maximum(h2 + b1_ref[...], 0.0)
        embd = jnp.concatenate([h2[0:1, :], h2[1:2, :]], axis=1)
        r = jnp.dot(embd, f1w_ref[...], preferred_element_type=jnp.float32)
        r = jnp.maximum(r + f1b_ref[...], 0.0)
        o_ref[...] = jnp.dot(r, f2w_ref[...],
                             preferred_element_type=jnp.float32) + f2b_ref[...]


_fin_tc = pl.pallas_call(
    _fin_body,
    grid=(_NPAD // _B,),
    in_specs=[
        pl.BlockSpec((2, _B), lambda k: (0, k)),
        pl.BlockSpec((2, _B), lambda k: (0, k)),
        pl.BlockSpec((_B, 16), lambda k: (k, 0)),
        pl.BlockSpec((16, 16), lambda k: (0, 0)),
        pl.BlockSpec((1, 16), lambda k: (0, 0)),
        pl.BlockSpec((32, 40), lambda k: (0, 0)),
        pl.BlockSpec((1, 40), lambda k: (0, 0)),
        pl.BlockSpec((40, 2), lambda k: (0, 0)),
        pl.BlockSpec((1, 2), lambda k: (0, 0)),
    ],
    out_specs=pl.BlockSpec((1, 2), lambda k: (0, 0)),
    out_shape=jax.ShapeDtypeStruct((1, 2), jnp.float32),
    scratch_shapes=[
        pltpu.VMEM((2, 16), jnp.float32),
        pltpu.VMEM((1, 2), jnp.float32),
    ],
)


def kernel(feature_torch, edge_torch, i, j, W0, b0, W1, b1, fc1_W, fc1_b,
           fc2_W, fc2_b):
    src = edge_torch[0]
    dst = edge_torch[1]
    pad = jnp.full((_EPAD - _E,), _DUMMY, jnp.int32)
    src2 = jnp.concatenate([src, pad]).reshape(_ROWS, 128)
    dst2 = jnp.concatenate([dst, pad]).reshape(_ROWS, 128)
    z1 = jnp.zeros((_NPAD,), jnp.float32)
    iv = jnp.full((16,), i, jnp.int32)
    jv = jnp.full((16,), j, jnp.int32)

    deg_part = _deg_sc(dst2, z1)
    xt = jnp.zeros((2, _NPAD), jnp.float32).at[:, :_N].set(feature_torch.T)
    y0, y1 = _y_tc(deg_part, xt)
    agg0_part, agg1_part, ci_part, cj_part = _agg_sc(
        src2, dst2, y0, y1, iv, jv, z1)
    h1s = _h1_tc(agg0_part, agg1_part, deg_part, W0, b0.reshape(1, 16))
    out = _fin_tc(ci_part, cj_part, h1s, W1, b1.reshape(1, 16),
                  fc1_W, fc1_b.reshape(1, 40), fc2_W, fc2_b.reshape(1, 2))
    return out.reshape(2)
